# baseline (speedup 1.0000x reference)
import functools

import jax
import jax.numpy as jnp
from jax import lax
from jax.experimental import pallas as pl
from jax.experimental.pallas import tpu as pltpu
from jax.experimental.pallas import tpu_sc as plsc

NC = 2
NS = 16
NW = NC * NS
CHUNK = 128
ZROWS = 128


def _cdiv(a, b):
    return (a + b - 1) // b


def _sc_mesh():
    return plsc.VectorSubcoreMesh(core_axis_name="c", subcore_axis_name="s",
                                  num_cores=NC, num_subcores=NS)


_SC_PARAMS = pltpu.CompilerParams(use_tc_tiling_on_sc=False)


def _deg_kernel(acc_rows, nch):
    DW = 16
    rows_per_tile = acc_rows // NS
    nz = rows_per_tile // ZROWS

    @functools.partial(
        pl.kernel,
        out_type=jax.ShapeDtypeStruct((NC, acc_rows, DW), jnp.float32),
        mesh=_sc_mesh(),
        scratch_types=[
            pltpu.VMEM((nch, CHUNK), jnp.int32),
            pltpu.VMEM((CHUNK, DW), jnp.float32),
            pltpu.VMEM((ZROWS, DW), jnp.float32),
            pltpu.VMEM_SHARED((acc_rows, DW), jnp.float32),
        ],
        compiler_params=_SC_PARAMS,
    )
    def k(dsts, out, dst_v, ones_v, zero_v, acc):
        cid = lax.axis_index("c")
        sid = lax.axis_index("s")
        wid = sid * NC + cid
        one16 = jnp.ones((16,), jnp.float32)
        zero16 = jnp.zeros((16,), jnp.float32)

        def fill(i, carry):
            ones_v[i, :] = one16
            zero_v[i, :] = zero16
            return carry

        lax.fori_loop(0, CHUNK, fill, 0)
        for b in range(nz):
            pltpu.sync_copy(
                zero_v, acc.at[pl.ds(sid * rows_per_tile + b * ZROWS, ZROWS)])
        pltpu.sync_copy(dsts.at[wid], dst_v)
        plsc.subcore_barrier()

        def body(j, carry):
            pltpu.sync_copy(ones_v, acc.at[dst_v.at[j]], add=True)
            return carry

        lax.fori_loop(0, nch, body, 0)
        plsc.subcore_barrier()
        for b in range(nz):
            off = sid * rows_per_tile + b * ZROWS
            pltpu.sync_copy(acc.at[pl.ds(off, ZROWS)],
                            out.at[cid, pl.ds(off, ZROWS)])

    return k


def _agg_kernel(acc_rows, nch, d):
    rows_per_tile = acc_rows // NS
    nz = rows_per_tile // ZROWS
    n16 = d // 16

    @functools.partial(
        pl.kernel,
        out_type=jax.ShapeDtypeStruct((NC, acc_rows, d), jnp.float32),
        mesh=_sc_mesh(),
        scratch_types=[
            pltpu.VMEM((nch, CHUNK), jnp.int32),
            pltpu.VMEM((nch, CHUNK), jnp.int32),
            pltpu.VMEM((CHUNK, d), jnp.float32),
            pltpu.VMEM((CHUNK, d), jnp.float32),
            pltpu.VMEM((ZROWS, d), jnp.float32),
            pltpu.VMEM_SHARED((acc_rows, d), jnp.float32),
            pltpu.SemaphoreType.DMA,
            pltpu.SemaphoreType.DMA,
            pltpu.SemaphoreType.DMA,
            pltpu.SemaphoreType.DMA,
        ],
        compiler_params=_SC_PARAMS,
    )
    def k(table, srcs, dsts, out, src_v, dst_v, rows0, rows1, zero_v, acc,
          gs0, gs1, ss0, ss1):
        cid = lax.axis_index("c")
        sid = lax.axis_index("s")
        wid = sid * NC + cid
        zero16 = jnp.zeros((16,), jnp.float32)

        def fill(i, carry):
            for t in range(n16):
                zero_v[i, pl.ds(t * 16, 16)] = zero16
            return carry

        lax.fori_loop(0, ZROWS, fill, 0)
        for b in range(nz):
            pltpu.sync_copy(
                zero_v, acc.at[pl.ds(sid * rows_per_tile + b * ZROWS, ZROWS)])
        pltpu.sync_copy(srcs.at[wid], src_v)
        pltpu.sync_copy(dsts.at[wid], dst_v)
        plsc.subcore_barrier()

        nloop = nch // 2
        pltpu.async_copy(table.at[src_v.at[0]], rows0, gs0)

        def body(i, carry):
            j0 = 2 * i
            j1 = j0 + 1
            pltpu.make_async_copy(table.at[src_v.at[j0]], rows0, gs0).wait()
            pltpu.async_copy(rows0, acc.at[dst_v.at[j0]], ss0, add=True)

            @pl.when(i > 0)
            def _():
                pltpu.make_async_copy(rows1, acc.at[dst_v.at[j0 - 1]],
                                      ss1).wait()

            pltpu.async_copy(table.at[src_v.at[j1]], rows1, gs1)
            pltpu.make_async_copy(table.at[src_v.at[j1]], rows1, gs1).wait()
            pltpu.async_copy(rows1, acc.at[dst_v.at[j1]], ss1, add=True)
            pltpu.make_async_copy(rows0, acc.at[dst_v.at[j0]], ss0).wait()

            @pl.when(i + 1 < nloop)
            def _():
                pltpu.async_copy(table.at[src_v.at[j0 + 2]], rows0, gs0)

            return carry

        lax.fori_loop(0, nloop, body, 0)
        pltpu.make_async_copy(rows1, acc.at[dst_v.at[nch - 1]], ss1).wait()
        plsc.subcore_barrier()
        for b in range(nz):
            off = sid * rows_per_tile + b * ZROWS
            pltpu.sync_copy(acc.at[pl.ds(off, ZROWS)],
                            out.at[cid, pl.ds(off, ZROWS)])

    return k


def _mm1_call(x, W1, degp):
    n = x.shape[0]
    d = W1.shape[1]

    def body(x_ref, w_ref, degp_ref, hs_ref, dinv_ref):
        deg = degp_ref[0, :n, 0:1] + degp_ref[1, :n, 0:1] + 1.0
        dinv = lax.rsqrt(deg)
        h = jnp.dot(x_ref[...], w_ref[...], preferred_element_type=jnp.float32)
        hs_ref[...] = h * dinv
        dinv_ref[...] = dinv

    return pl.pallas_call(
        body,
        out_shape=(jax.ShapeDtypeStruct((n, d), jnp.float32),
                   jax.ShapeDtypeStruct((n, 1), jnp.float32)),
    )(x, W1, degp)


def _mid_call(p, hs, dinv, b1, gamma, beta, W2):
    n, d = hs.shape

    def body(p_ref, hs_ref, dinv_ref, b1_ref, g_ref, be_ref, w2_ref, gs_ref):
        dinv_v = dinv_ref[...]
        agg = p_ref[0, :n, :] + p_ref[1, :n, :] + hs_ref[...]
        h1 = agg * dinv_v + b1_ref[...]
        mean = jnp.mean(h1, axis=0, keepdims=True)
        cent = h1 - mean
        var = jnp.mean(cent * cent, axis=0, keepdims=True)
        h2 = jnp.maximum(
            g_ref[...] * cent * lax.rsqrt(var + 1e-5) + be_ref[...], 0.0)
        g2 = jnp.dot(h2, w2_ref[...], preferred_element_type=jnp.float32)
        gs_ref[...] = g2 * dinv_v

    return pl.pallas_call(
        body,
        out_shape=jax.ShapeDtypeStruct((n, d), jnp.float32),
    )(p, hs, dinv, b1, gamma, beta, W2)


def _out_call(p, gs, dinv, b2):
    n, d = gs.shape

    def body(p_ref, gs_ref, dinv_ref, b2_ref, out_ref):
        agg = p_ref[0, :n, :] + p_ref[1, :n, :] + gs_ref[...]
        out_ref[...] = agg * dinv_ref[...] + b2_ref[...]

    return pl.pallas_call(
        body,
        out_shape=jax.ShapeDtypeStruct((n, d), jnp.float32),
    )(p, gs, dinv, b2)


def kernel(x, edge_index, W1, b1, gamma, beta, W2, b2):
    n = x.shape[0]
    e = edge_index.shape[1]
    d = W1.shape[1]

    src = edge_index[0].astype(jnp.int32)
    dst = edge_index[1].astype(jnp.int32)

    nch = _cdiv(_cdiv(e, NW * CHUNK), 2) * 2
    epw = nch * CHUNK
    pad = epw * NW - e
    src_p = jnp.concatenate([src, jnp.zeros((pad,), jnp.int32)])
    dst_p = jnp.concatenate([dst, jnp.full((pad,), n, jnp.int32)])
    srcs = src_p.reshape(NW, nch, CHUNK)
    dsts = dst_p.reshape(NW, nch, CHUNK)

    acc_rows = _cdiv(n + 1, NS * ZROWS) * NS * ZROWS

    degp = _deg_kernel(acc_rows, nch)(dsts)
    hs, dinv = _mm1_call(x, W1, degp)

    agg = _agg_kernel(acc_rows, nch, d)
    p1 = agg(hs, srcs, dsts)
    gs = _mid_call(p1, hs, dinv, b1.reshape(1, d), gamma.reshape(1, d),
                   beta.reshape(1, d), W2)
    p2 = agg(gs, srcs, dsts)
    return _out_call(p2, gs, dinv, b2.reshape(1, d))


# speedup vs baseline: 1.7388x; 1.7388x over previous
import functools

import jax
import jax.numpy as jnp
from jax import lax
from jax.experimental import pallas as pl
from jax.experimental.pallas import tpu as pltpu
from jax.experimental.pallas import tpu_sc as plsc

NC = 2
NS = 16
NW = NC * NS
CHUNK = 128
ZROWS = 128


def _cdiv(a, b):
    return (a + b - 1) // b


def _sc_mesh():
    return plsc.VectorSubcoreMesh(core_axis_name="c", subcore_axis_name="s",
                                  num_cores=NC, num_subcores=NS)


_SC_PARAMS = pltpu.CompilerParams(use_tc_tiling_on_sc=False)


def _deg_kernel(acc_rows, nch):
    DW = 16
    rows_per_tile = acc_rows // NS
    nz = rows_per_tile // ZROWS

    @functools.partial(
        pl.kernel,
        out_type=jax.ShapeDtypeStruct((NC, acc_rows, DW), jnp.float32),
        mesh=_sc_mesh(),
        scratch_types=[
            pltpu.VMEM((nch, CHUNK), jnp.int32),
            pltpu.VMEM((CHUNK, DW), jnp.float32),
            pltpu.VMEM((ZROWS, DW), jnp.float32),
            pltpu.VMEM_SHARED((acc_rows, DW), jnp.float32),
        ],
        compiler_params=_SC_PARAMS,
    )
    def k(dsts, out, dst_v, ones_v, zero_v, acc):
        cid = lax.axis_index("c")
        sid = lax.axis_index("s")
        wid = sid * NC + cid
        one16 = jnp.ones((16,), jnp.float32)
        zero16 = jnp.zeros((16,), jnp.float32)

        def fill(i, carry):
            ones_v[i, :] = one16
            zero_v[i, :] = zero16
            return carry

        lax.fori_loop(0, CHUNK, fill, 0)
        for b in range(nz):
            pltpu.sync_copy(
                zero_v, acc.at[pl.ds(sid * rows_per_tile + b * ZROWS, ZROWS)])
        pltpu.sync_copy(dsts.at[wid], dst_v)
        plsc.subcore_barrier()

        def body(j, carry):
            pltpu.sync_copy(ones_v, acc.at[dst_v.at[j]], add=True)
            return carry

        lax.fori_loop(0, nch, body, 0)
        plsc.subcore_barrier()
        for b in range(nz):
            off = sid * rows_per_tile + b * ZROWS
            pltpu.sync_copy(acc.at[pl.ds(off, ZROWS)],
                            out.at[cid, pl.ds(off, ZROWS)])

    return k


def _agg_kernel(acc_rows, nch, d):
    rows_per_tile = acc_rows // NS
    nz = rows_per_tile // ZROWS
    n16 = d // 16

    @functools.partial(
        pl.kernel,
        out_type=jax.ShapeDtypeStruct((NC, acc_rows, d), jnp.float32),
        mesh=_sc_mesh(),
        scratch_types=[
            pltpu.VMEM((nch, CHUNK), jnp.int32),
            pltpu.VMEM((nch, CHUNK), jnp.int32),
            pltpu.VMEM((CHUNK, d), jnp.float32),
            pltpu.VMEM((ZROWS, d), jnp.float32),
            pltpu.VMEM_SHARED((acc_rows, d), jnp.float32),
            pltpu.VMEM_SHARED((acc_rows, d), jnp.float32),
            pltpu.SemaphoreType.DMA,
        ],
        compiler_params=_SC_PARAMS,
    )
    def k(table, srcs, dsts, out, src_v, dst_v, rows_v, zero_v, acc,
          table_sh, sem):
        cid = lax.axis_index("c")
        sid = lax.axis_index("s")
        wid = sid * NC + cid
        zero16 = jnp.zeros((16,), jnp.float32)

        def fill(i, carry):
            for t in range(n16):
                zero_v[i, pl.ds(t * 16, 16)] = zero16
            return carry

        lax.fori_loop(0, ZROWS, fill, 0)
        for b in range(nz):
            pltpu.sync_copy(
                zero_v, acc.at[pl.ds(sid * rows_per_tile + b * ZROWS, ZROWS)])
        n_nodes = table.shape[0]
        stage = n_nodes // NS
        pltpu.sync_copy(table.at[pl.ds(sid * stage, stage)],
                        table_sh.at[pl.ds(sid * stage, stage)])
        pltpu.sync_copy(srcs.at[wid], src_v)
        pltpu.sync_copy(dsts.at[wid], dst_v)
        plsc.subcore_barrier()

        def body(j, carry):
            pltpu.sync_copy(table_sh.at[src_v.at[j]], rows_v)
            pltpu.sync_copy(rows_v, acc.at[dst_v.at[j]], add=True)
            return carry

        lax.fori_loop(0, nch, body, 0)
        plsc.subcore_barrier()
        for b in range(nz):
            off = sid * rows_per_tile + b * ZROWS
            pltpu.sync_copy(acc.at[pl.ds(off, ZROWS)],
                            out.at[cid, pl.ds(off, ZROWS)])

    return k


def _mm1_call(x, W1, degp):
    n = x.shape[0]
    d = W1.shape[1]

    def body(x_ref, w_ref, degp_ref, hs_ref, dinv_ref):
        deg = degp_ref[0, :n, 0:1] + degp_ref[1, :n, 0:1] + 1.0
        dinv = lax.rsqrt(deg)
        h = jnp.dot(x_ref[...], w_ref[...], preferred_element_type=jnp.float32)
        hs_ref[...] = h * dinv
        dinv_ref[...] = dinv

    return pl.pallas_call(
        body,
        out_shape=(jax.ShapeDtypeStruct((n, d), jnp.float32),
                   jax.ShapeDtypeStruct((n, 1), jnp.float32)),
    )(x, W1, degp)


def _mid_call(p, hs, dinv, b1, gamma, beta, W2):
    n, d = hs.shape

    def body(p_ref, hs_ref, dinv_ref, b1_ref, g_ref, be_ref, w2_ref, gs_ref):
        dinv_v = dinv_ref[...]
        agg = p_ref[0, :n, :] + p_ref[1, :n, :] + hs_ref[...]
        h1 = agg * dinv_v + b1_ref[...]
        mean = jnp.mean(h1, axis=0, keepdims=True)
        cent = h1 - mean
        var = jnp.mean(cent * cent, axis=0, keepdims=True)
        h2 = jnp.maximum(
            g_ref[...] * cent * lax.rsqrt(var + 1e-5) + be_ref[...], 0.0)
        g2 = jnp.dot(h2, w2_ref[...], preferred_element_type=jnp.float32)
        gs_ref[...] = g2 * dinv_v

    return pl.pallas_call(
        body,
        out_shape=jax.ShapeDtypeStruct((n, d), jnp.float32),
    )(p, hs, dinv, b1, gamma, beta, W2)


def _out_call(p, gs, dinv, b2):
    n, d = gs.shape

    def body(p_ref, gs_ref, dinv_ref, b2_ref, out_ref):
        agg = p_ref[0, :n, :] + p_ref[1, :n, :] + gs_ref[...]
        out_ref[...] = agg * dinv_ref[...] + b2_ref[...]

    return pl.pallas_call(
        body,
        out_shape=jax.ShapeDtypeStruct((n, d), jnp.float32),
    )(p, gs, dinv, b2)


def kernel(x, edge_index, W1, b1, gamma, beta, W2, b2):
    n = x.shape[0]
    e = edge_index.shape[1]
    d = W1.shape[1]

    src = edge_index[0].astype(jnp.int32)
    dst = edge_index[1].astype(jnp.int32)

    nch = _cdiv(_cdiv(e, NW * CHUNK), 2) * 2
    epw = nch * CHUNK
    pad = epw * NW - e
    src_p = jnp.concatenate([src, jnp.zeros((pad,), jnp.int32)])
    dst_p = jnp.concatenate([dst, jnp.full((pad,), n, jnp.int32)])
    srcs = src_p.reshape(NW, nch, CHUNK)
    dsts = dst_p.reshape(NW, nch, CHUNK)

    acc_rows = _cdiv(n + 1, NS * ZROWS) * NS * ZROWS

    degp = _deg_kernel(acc_rows, nch)(dsts)
    hs, dinv = _mm1_call(x, W1, degp)

    agg = _agg_kernel(acc_rows, nch, d)
    p1 = agg(hs, srcs, dsts)
    gs = _mid_call(p1, hs, dinv, b1.reshape(1, d), gamma.reshape(1, d),
                   beta.reshape(1, d), W2)
    p2 = agg(gs, srcs, dsts)
    return _out_call(p2, gs, dinv, b2.reshape(1, d))


# speedup vs baseline: 2.0037x; 1.1524x over previous
import functools

import jax
import jax.numpy as jnp
from jax import lax
from jax.experimental import pallas as pl
from jax.experimental.pallas import tpu as pltpu
from jax.experimental.pallas import tpu_sc as plsc

NC = 2
NS = 16
NW = NC * NS
CHUNK = 128
ZROWS = 128
DW = 8


def _cdiv(a, b):
    return (a + b - 1) // b


def _sc_mesh():
    return plsc.VectorSubcoreMesh(core_axis_name="c", subcore_axis_name="s",
                                  num_cores=NC, num_subcores=NS)


_SC_PARAMS = pltpu.CompilerParams(use_tc_tiling_on_sc=False)


def _deg_kernel(acc_rows, nch):
    rows_per_tile = acc_rows // NS
    nz = rows_per_tile // ZROWS

    @functools.partial(
        pl.kernel,
        out_type=jax.ShapeDtypeStruct((NC, acc_rows, DW), jnp.float32),
        mesh=_sc_mesh(),
        scratch_types=[
            pltpu.VMEM((nch, CHUNK), jnp.int32),
            pltpu.VMEM((CHUNK, DW), jnp.float32),
            pltpu.VMEM_SHARED((acc_rows, DW), jnp.float32),
        ],
        compiler_params=_SC_PARAMS,
    )
    def k(ep, ones8, zer8, out, dst_v, ones_v, acc):
        cid = lax.axis_index("c")
        sid = lax.axis_index("s")
        wid = sid * NC + cid
        for b in range(nz):
            pltpu.sync_copy(
                zer8, acc.at[pl.ds(sid * rows_per_tile + b * ZROWS, ZROWS)])
        pltpu.sync_copy(ones8, ones_v)
        pltpu.sync_copy(ep.at[1, wid], dst_v)
        plsc.subcore_barrier()

        def body(j, carry):
            pltpu.sync_copy(ones_v, acc.at[dst_v.at[j]], add=True)
            return carry

        lax.fori_loop(0, nch, body, 0)
        plsc.subcore_barrier()
        for b in range(nz):
            off = sid * rows_per_tile + b * ZROWS
            pltpu.sync_copy(acc.at[pl.ds(off, ZROWS)],
                            out.at[cid, pl.ds(off, ZROWS)])

    return k


def _agg_kernel(acc_rows, nch, d):
    rows_per_tile = acc_rows // NS
    nz = rows_per_tile // ZROWS
    n16 = d // 16

    @functools.partial(
        pl.kernel,
        out_type=jax.ShapeDtypeStruct((NC, acc_rows, 128), jnp.float32),
        mesh=_sc_mesh(),
        scratch_types=[
            pltpu.VMEM((nch, CHUNK), jnp.int32),
            pltpu.VMEM((nch, CHUNK), jnp.int32),
            pltpu.VMEM((CHUNK, d), jnp.float32),
            pltpu.VMEM((ZROWS, d), jnp.float32),
            pltpu.VMEM((ZROWS, d), jnp.float32),
            pltpu.VMEM_SHARED((acc_rows, d), jnp.float32),
            pltpu.VMEM_SHARED((acc_rows, d), jnp.float32),
            pltpu.SemaphoreType.DMA,
        ],
        compiler_params=_SC_PARAMS,
    )
    def k(table, ep, out, src_v, dst_v, rows_v, buf, zero_v, acc, table_sh,
          sem):
        cid = lax.axis_index("c")
        sid = lax.axis_index("s")
        wid = sid * NC + cid
        zero16 = jnp.zeros((16,), jnp.float32)

        def fill(i, carry):
            for t in range(n16):
                zero_v[i, pl.ds(t * 16, 16)] = zero16
            return carry

        lax.fori_loop(0, ZROWS, fill, 0)
        for b in range(nz):
            off = sid * rows_per_tile + b * ZROWS
            pltpu.sync_copy(table.at[pl.ds(off, ZROWS), pl.ds(0, d)], buf)
            pltpu.sync_copy(buf, table_sh.at[pl.ds(off, ZROWS)])

            @pl.when(cid == 0)
            def _():
                pltpu.sync_copy(buf, acc.at[pl.ds(off, ZROWS)])

            @pl.when(cid != 0)
            def _():
                pltpu.sync_copy(zero_v, acc.at[pl.ds(off, ZROWS)])

        pltpu.sync_copy(ep.at[0, wid], src_v)
        pltpu.sync_copy(ep.at[1, wid], dst_v)
        plsc.subcore_barrier()

        def body(j, carry):
            pltpu.sync_copy(table_sh.at[src_v.at[j]], rows_v)
            pltpu.sync_copy(rows_v, acc.at[dst_v.at[j]], add=True)
            return carry

        lax.fori_loop(0, nch, body, 0)
        plsc.subcore_barrier()
        for b in range(nz):
            off = sid * rows_per_tile + b * ZROWS
            pltpu.sync_copy(acc.at[pl.ds(off, ZROWS)],
                            out.at[cid, pl.ds(off, ZROWS), pl.ds(0, d)])

    return k


def _mm1_call(x, W1, degp, acc_rows):
    n = x.shape[0]
    d = W1.shape[1]

    def body(x_ref, w_ref, dg_ref, hs_ref, dinv_ref):
        deg = dg_ref[0, :n, 0:1] + dg_ref[1, :n, 0:1] + 1.0
        dinv = lax.rsqrt(deg)
        h = jnp.dot(x_ref[...], w_ref[...], preferred_element_type=jnp.float32)
        hs_ref[0:n, :] = jnp.concatenate(
            [h * dinv, jnp.zeros((n, 128 - d), jnp.float32)], axis=1)
        hs_ref[n:acc_rows, :] = jnp.zeros((acc_rows - n, 128), jnp.float32)
        dinv_ref[...] = dinv

    return pl.pallas_call(
        body,
        out_shape=(jax.ShapeDtypeStruct((acc_rows, 128), jnp.float32),
                   jax.ShapeDtypeStruct((n, 1), jnp.float32)),
    )(x, W1, degp)


def _mid_call(p, dinv, b1, gamma, beta, W2, acc_rows):
    n = dinv.shape[0]
    d = W2.shape[0]

    def body(p_ref, dinv_ref, b1_ref, g_ref, be_ref, w2_ref, gs_ref):
        dinv_v = dinv_ref[...]
        agg = p_ref[0, :n, 0:d] + p_ref[1, :n, 0:d]
        h1 = agg * dinv_v + b1_ref[...]
        mean = jnp.mean(h1, axis=0, keepdims=True)
        cent = h1 - mean
        var = jnp.mean(cent * cent, axis=0, keepdims=True)
        h2 = jnp.maximum(
            g_ref[...] * cent * lax.rsqrt(var + 1e-5) + be_ref[...], 0.0)
        g2 = jnp.dot(h2, w2_ref[...], preferred_element_type=jnp.float32)
        gs_ref[0:n, :] = jnp.concatenate(
            [g2 * dinv_v, jnp.zeros((n, 128 - d), jnp.float32)], axis=1)
        gs_ref[n:acc_rows, :] = jnp.zeros((acc_rows - n, 128), jnp.float32)

    return pl.pallas_call(
        body,
        out_shape=jax.ShapeDtypeStruct((acc_rows, 128), jnp.float32),
    )(p, dinv, b1, gamma, beta, W2)


def _out_call(p, dinv, b2):
    n = dinv.shape[0]
    d = b2.shape[1]

    def body(p_ref, dinv_ref, b2_ref, out_ref):
        agg = p_ref[0, :n, 0:d] + p_ref[1, :n, 0:d]
        out_ref[...] = agg * dinv_ref[...] + b2_ref[...]

    return pl.pallas_call(
        body,
        out_shape=jax.ShapeDtypeStruct((n, d), jnp.float32),
    )(p, dinv, b2)


def kernel(x, edge_index, W1, b1, gamma, beta, W2, b2):
    n = x.shape[0]
    e = edge_index.shape[1]
    d = W1.shape[1]

    nch = _cdiv(e, NW * CHUNK)
    pad = nch * CHUNK * NW - e
    ep = jnp.pad(edge_index.astype(jnp.int32), ((0, 0), (0, pad)),
                 constant_values=n).reshape(2, NW, nch, CHUNK)

    acc_rows = _cdiv(n + 1, NS * ZROWS) * NS * ZROWS

    ones8 = jnp.ones((CHUNK, DW), jnp.float32)
    zer8 = jnp.zeros((ZROWS, DW), jnp.float32)
    degp = _deg_kernel(acc_rows, nch)(ep, ones8, zer8)
    hs, dinv = _mm1_call(x, W1, degp, acc_rows)

    agg = _agg_kernel(acc_rows, nch, d)
    p1 = agg(hs, ep)
    gs = _mid_call(p1, dinv, b1.reshape(1, d), gamma.reshape(1, d),
                   beta.reshape(1, d), W2, acc_rows)
    p2 = agg(gs, ep)
    return _out_call(p2, dinv, b2.reshape(1, d))


# speedup vs baseline: 2.3720x; 1.1838x over previous
import functools

import jax
import jax.numpy as jnp
from jax import lax
from jax.experimental import pallas as pl
from jax.experimental.pallas import tpu as pltpu
from jax.experimental.pallas import tpu_sc as plsc

NC = 2
NS = 16
NW = NC * NS
CHUNK = 128
ZROWS = 128
DW = 8
NBUF = 4


def _cdiv(a, b):
    return (a + b - 1) // b


def _sc_mesh():
    return plsc.VectorSubcoreMesh(core_axis_name="c", subcore_axis_name="s",
                                  num_cores=NC, num_subcores=NS)


_SC_PARAMS = pltpu.CompilerParams(use_tc_tiling_on_sc=False)


def _deg_kernel(acc_rows, nch):
    rows_per_tile = acc_rows // NS
    nz = rows_per_tile // ZROWS

    @functools.partial(
        pl.kernel,
        out_type=jax.ShapeDtypeStruct((NC, acc_rows, 128), jnp.float32),
        mesh=_sc_mesh(),
        scratch_types=[
            pltpu.VMEM((nch, CHUNK), jnp.int32),
            pltpu.VMEM((CHUNK, DW), jnp.float32),
            pltpu.VMEM_SHARED((acc_rows, DW), jnp.float32),
            pltpu.SemaphoreType.DMA,
        ],
        compiler_params=_SC_PARAMS,
    )
    def k(ep, ones8, zer8, out, dst_v, ones_v, acc, dsem):
        cid = lax.axis_index("c")
        sid = lax.axis_index("s")
        wid = sid * NC + cid
        for b in range(nz):
            pltpu.sync_copy(
                zer8, acc.at[pl.ds(sid * rows_per_tile + b * ZROWS, ZROWS)])
        pltpu.sync_copy(ones8, ones_v)
        pltpu.sync_copy(ep.at[1, wid], dst_v)
        plsc.subcore_barrier()

        def body(j, carry):
            pltpu.async_copy(ones_v, acc.at[dst_v.at[j]], dsem, add=True)
            return carry

        lax.fori_loop(0, nch, body, 0)

        def drain(j, carry):
            pltpu.make_async_copy(ones_v, acc.at[dst_v.at[j]], dsem).wait()
            return carry

        lax.fori_loop(0, nch, drain, 0)
        plsc.subcore_barrier()
        for b in range(nz):
            off = sid * rows_per_tile + b * ZROWS
            pltpu.sync_copy(acc.at[pl.ds(off, ZROWS)],
                            out.at[cid, pl.ds(off, ZROWS), pl.ds(0, DW)])

    return k


def _agg_kernel(acc_rows, nch, d):
    rows_per_tile = acc_rows // NS
    nz = rows_per_tile // ZROWS
    n16 = d // 16

    @functools.partial(
        pl.kernel,
        out_type=jax.ShapeDtypeStruct((NC, acc_rows, 128), jnp.float32),
        mesh=_sc_mesh(),
        scratch_types=[
            pltpu.VMEM((nch, CHUNK), jnp.int32),
            pltpu.VMEM((nch, CHUNK), jnp.int32),
            pltpu.VMEM((NBUF, CHUNK, d), jnp.float32),
            pltpu.VMEM((ZROWS, d), jnp.float32),
            pltpu.VMEM((ZROWS, d), jnp.float32),
            pltpu.VMEM_SHARED((acc_rows, d), jnp.float32),
            pltpu.VMEM_SHARED((acc_rows, d), jnp.float32),
        ] + [pltpu.SemaphoreType.DMA] * (2 * NBUF),
        compiler_params=_SC_PARAMS,
    )
    def k(table, ep, out, src_v, dst_v, rows_v, buf, zero_v, acc, table_sh,
          *sems):
        gsem = sems[:NBUF]
        ssem = sems[NBUF:]
        cid = lax.axis_index("c")
        sid = lax.axis_index("s")
        wid = sid * NC + cid
        zero16 = jnp.zeros((16,), jnp.float32)

        def fill(i, carry):
            for t in range(n16):
                zero_v[i, pl.ds(t * 16, 16)] = zero16
            return carry

        lax.fori_loop(0, ZROWS, fill, 0)
        for b in range(nz):
            off = sid * rows_per_tile + b * ZROWS
            pltpu.sync_copy(table.at[pl.ds(off, ZROWS), pl.ds(0, d)], buf)
            pltpu.sync_copy(buf, table_sh.at[pl.ds(off, ZROWS)])

            @pl.when(cid == 0)
            def _():
                pltpu.sync_copy(buf, acc.at[pl.ds(off, ZROWS)])

            @pl.when(cid != 0)
            def _():
                pltpu.sync_copy(zero_v, acc.at[pl.ds(off, ZROWS)])

        pltpu.sync_copy(ep.at[0, wid], src_v)
        pltpu.sync_copy(ep.at[1, wid], dst_v)
        plsc.subcore_barrier()

        for b in range(NBUF):
            pltpu.async_copy(table_sh.at[src_v.at[b]], rows_v.at[b], gsem[b])

        def body(i, carry):
            for b in range(NBUF):
                j = NBUF * i + b
                pltpu.make_async_copy(table_sh.at[src_v.at[j]], rows_v.at[b],
                                      gsem[b]).wait()
                pltpu.async_copy(rows_v.at[b], acc.at[dst_v.at[j]], ssem[b],
                                 add=True)

                @pl.when(j + NBUF < nch)
                def _():
                    pltpu.make_async_copy(rows_v.at[b], acc.at[dst_v.at[j]],
                                          ssem[b]).wait()
                    pltpu.async_copy(table_sh.at[src_v.at[j + NBUF]],
                                     rows_v.at[b], gsem[b])

            return carry

        lax.fori_loop(0, nch // NBUF, body, 0)
        for b in range(NBUF):
            j = nch - NBUF + b
            pltpu.make_async_copy(rows_v.at[b], acc.at[dst_v.at[j]],
                                  ssem[b]).wait()
        plsc.subcore_barrier()
        for b in range(nz):
            off = sid * rows_per_tile + b * ZROWS
            pltpu.sync_copy(acc.at[pl.ds(off, ZROWS)],
                            out.at[cid, pl.ds(off, ZROWS), pl.ds(0, d)])

    return k


def _mm1_call(x, W1, degp, acc_rows):
    n = x.shape[0]
    d = W1.shape[1]

    def body(x_ref, w_ref, dg_ref, hs_ref, dinv_ref):
        deg = dg_ref[0, :n, 0:1] + dg_ref[1, :n, 0:1] + 1.0
        dinv = lax.rsqrt(deg)
        h = jnp.dot(x_ref[...], w_ref[...], preferred_element_type=jnp.float32)
        hs_ref[0:n, :] = jnp.concatenate(
            [h * dinv, jnp.zeros((n, 128 - d), jnp.float32)], axis=1)
        hs_ref[n:acc_rows, :] = jnp.zeros((acc_rows - n, 128), jnp.float32)
        dinv_ref[...] = dinv

    return pl.pallas_call(
        body,
        out_shape=(jax.ShapeDtypeStruct((acc_rows, 128), jnp.float32),
                   jax.ShapeDtypeStruct((n, 1), jnp.float32)),
    )(x, W1, degp)


def _mid_call(p, dinv, b1, gamma, beta, W2, acc_rows):
    n = dinv.shape[0]
    d = W2.shape[0]

    def body(p_ref, dinv_ref, b1_ref, g_ref, be_ref, w2_ref, gs_ref):
        dinv_v = dinv_ref[...]
        agg = p_ref[0, :n, 0:d] + p_ref[1, :n, 0:d]
        h1 = agg * dinv_v + b1_ref[...]
        mean = jnp.mean(h1, axis=0, keepdims=True)
        cent = h1 - mean
        var = jnp.mean(cent * cent, axis=0, keepdims=True)
        h2 = jnp.maximum(
            g_ref[...] * cent * lax.rsqrt(var + 1e-5) + be_ref[...], 0.0)
        g2 = jnp.dot(h2, w2_ref[...], preferred_element_type=jnp.float32)
        gs_ref[0:n, :] = jnp.concatenate(
            [g2 * dinv_v, jnp.zeros((n, 128 - d), jnp.float32)], axis=1)
        gs_ref[n:acc_rows, :] = jnp.zeros((acc_rows - n, 128), jnp.float32)

    return pl.pallas_call(
        body,
        out_shape=jax.ShapeDtypeStruct((acc_rows, 128), jnp.float32),
    )(p, dinv, b1, gamma, beta, W2)


def _out_call(p, dinv, b2):
    n = dinv.shape[0]
    d = b2.shape[1]

    def body(p_ref, dinv_ref, b2_ref, out_ref):
        agg = p_ref[0, :n, 0:d] + p_ref[1, :n, 0:d]
        out_ref[...] = agg * dinv_ref[...] + b2_ref[...]

    return pl.pallas_call(
        body,
        out_shape=jax.ShapeDtypeStruct((n, d), jnp.float32),
    )(p, dinv, b2)


def kernel(x, edge_index, W1, b1, gamma, beta, W2, b2):
    n = x.shape[0]
    e = edge_index.shape[1]
    d = W1.shape[1]

    nch = _cdiv(_cdiv(e, NW * CHUNK), NBUF) * NBUF
    pad = nch * CHUNK * NW - e
    ep = jnp.pad(edge_index.astype(jnp.int32), ((0, 0), (0, pad)),
                 constant_values=n).reshape(2, NW, nch, CHUNK)

    acc_rows = _cdiv(n + 1, NS * ZROWS) * NS * ZROWS

    ones8 = jnp.ones((CHUNK, DW), jnp.float32)
    zer8 = jnp.zeros((ZROWS, DW), jnp.float32)
    degp = _deg_kernel(acc_rows, nch)(ep, ones8, zer8)
    hs, dinv = _mm1_call(x, W1, degp, acc_rows)

    agg = _agg_kernel(acc_rows, nch, d)
    p1 = agg(hs, ep)
    gs = _mid_call(p1, dinv, b1.reshape(1, d), gamma.reshape(1, d),
                   beta.reshape(1, d), W2, acc_rows)
    p2 = agg(gs, ep)
    return _out_call(p2, dinv, b2.reshape(1, d))
